# baseline (speedup 1.0000x reference)
import functools

import jax
import jax.numpy as jnp
from jax.experimental import pallas as pl
from jax.experimental.pallas import tpu as pltpu

D = 768
E = 8
DR = 64
TEMP = 0.7
TT = 256


def _silu(v):
    return v * jax.nn.sigmoid(v)


def _layernorm(v, g, b):
    m = jnp.mean(v, axis=-1, keepdims=True)
    var = jnp.var(v, axis=-1, keepdims=True)
    return (v - m) * jax.lax.rsqrt(var + 1e-5) * g + b


def _ln_router_kernel(x_ref, g_ref, b_ref, wr_ref, cb_ref, t_ref, gates_ref):
    t = _layernorm(x_ref[...], g_ref[...], b_ref[...])
    t_ref[...] = t
    z = jax.lax.dot_general(t, wr_ref[...], (((1,), (1,)), ((), ())),
                            preferred_element_type=jnp.float32)
    logits = jax.lax.dot_general(z, cb_ref[...], (((1,), (1,)), ((), ())),
                                 preferred_element_type=jnp.float32) / TEMP
    lm = jnp.max(logits, axis=-1, keepdims=True)
    p = jnp.exp(logits - lm)
    gates = p / jnp.sum(p, axis=-1, keepdims=True)
    T = gates.shape[0]
    iota_e = jax.lax.broadcasted_iota(jnp.int32, (T, E), 1)
    m1 = jnp.max(gates, axis=-1, keepdims=True)
    i1 = jnp.min(jnp.where(gates == m1, iota_e, E), axis=-1, keepdims=True)
    neg = jnp.float32(-jnp.inf)
    masked = jnp.where(iota_e == i1, neg, gates)
    m2 = jnp.max(masked, axis=-1, keepdims=True)
    i2 = jnp.min(jnp.where(masked == m2, iota_e, E), axis=-1, keepdims=True)
    s = m1 + m2
    gates_ref[...] = (jnp.where(iota_e == i1, m1 / s, 0.0)
                      + jnp.where(iota_e == i2, m2 / s, 0.0))


def _experts_kernel(t_ref, gates_ref, w1_ref, b1_ref, w2_ref, b2_ref,
                    moe_ref):
    e = pl.program_id(0)
    i = pl.program_id(1)
    tb = t_ref[...].astype(jnp.bfloat16)
    w1 = w1_ref[0]
    a = jax.lax.dot_general(tb, w1, (((1,), (1,)), ((), ())),
                            preferred_element_type=jnp.float32)
    a = _silu(a + b1_ref[0])
    ab = a.astype(jnp.bfloat16)
    w2 = w2_ref[0]
    eo = jax.lax.dot_general(ab, w2, (((1,), (1,)), ((), ())),
                             preferred_element_type=jnp.float32)
    eo = eo + b2_ref[0]
    iota_e = jax.lax.broadcasted_iota(jnp.int32, gates_ref.shape, 1)
    g = jnp.sum(jnp.where(iota_e == e, gates_ref[...], 0.0), axis=1,
                keepdims=True)
    sl = pl.ds(i * TT, TT)

    @pl.when(e == 0)
    def _():
        moe_ref[sl, :] = g * eo

    @pl.when(e > 0)
    def _():
        moe_ref[sl, :] = moe_ref[sl, :] + g * eo


def _tail_kernel(x_ref, t_ref, moe_ref, ws1_ref, bs1_ref, ws2_ref, bs2_ref,
                 g2_ref, b2_ref, wf1_ref, bf1_ref, wf2_ref, bf2_ref, o_ref):
    tb = t_ref[...].astype(jnp.bfloat16)
    s1 = jax.lax.dot_general(tb, ws1_ref[...], (((1,), (1,)), ((), ())),
                             preferred_element_type=jnp.float32)
    s1 = _silu(s1 + bs1_ref[...]).astype(jnp.bfloat16)
    shared = jax.lax.dot_general(s1, ws2_ref[...], (((1,), (1,)), ((), ())),
                                 preferred_element_type=jnp.float32)
    shared = (shared + bs2_ref[...]) * 0.25
    hm = moe_ref[...] + shared
    f = _layernorm(hm, g2_ref[...], b2_ref[...]).astype(jnp.bfloat16)
    f1 = jax.lax.dot_general(f, wf1_ref[...], (((1,), (1,)), ((), ())),
                             preferred_element_type=jnp.float32)
    f1 = _silu(f1 + bf1_ref[...]).astype(jnp.bfloat16)
    f2 = jax.lax.dot_general(f1, wf2_ref[...], (((1,), (1,)), ((), ())),
                             preferred_element_type=jnp.float32)
    f2 = f2 + bf2_ref[...]
    o_ref[...] = x_ref[...] + hm + f2


def kernel(x, ln1_g, ln1_b, Wr, codebook, We1, be1, We2, be2,
           Ws1, bs1, Ws2, bs2, ln2_g, ln2_b, Wf1, bf1, Wf2, bf2):
    B, S, Dx = x.shape
    T = B * S
    xf = x.reshape(T, Dx)
    nt = T // TT

    t, gates = pl.pallas_call(
        _ln_router_kernel,
        out_shape=(jax.ShapeDtypeStruct((T, D), jnp.float32),
                   jax.ShapeDtypeStruct((T, E), jnp.float32)),
    )(xf, ln1_g.reshape(1, D), ln1_b.reshape(1, D), Wr, codebook)

    We1b = We1.astype(jnp.bfloat16)
    We2b = We2.astype(jnp.bfloat16)
    moe = pl.pallas_call(
        _experts_kernel,
        grid=(E, nt),
        in_specs=[
            pl.BlockSpec((TT, D), lambda e, i: (i, 0)),
            pl.BlockSpec((TT, E), lambda e, i: (i, 0)),
            pl.BlockSpec((1, 4 * D, D), lambda e, i: (e, 0, 0)),
            pl.BlockSpec((1, 1, 4 * D), lambda e, i: (e, 0, 0)),
            pl.BlockSpec((1, D, 4 * D), lambda e, i: (e, 0, 0)),
            pl.BlockSpec((1, 1, D), lambda e, i: (e, 0, 0)),
        ],
        out_specs=pl.BlockSpec((T, D), lambda e, i: (0, 0)),
        out_shape=jax.ShapeDtypeStruct((T, D), jnp.float32),
    )(t, gates, We1b, be1.reshape(E, 1, 4 * D), We2b, be2.reshape(E, 1, D))

    out = pl.pallas_call(
        _tail_kernel,
        grid=(nt,),
        in_specs=[
            pl.BlockSpec((TT, D), lambda i: (i, 0)),
            pl.BlockSpec((TT, D), lambda i: (i, 0)),
            pl.BlockSpec((TT, D), lambda i: (i, 0)),
            pl.BlockSpec((2 * D, D), lambda i: (0, 0)),
            pl.BlockSpec((1, 2 * D), lambda i: (0, 0)),
            pl.BlockSpec((D, 2 * D), lambda i: (0, 0)),
            pl.BlockSpec((1, D), lambda i: (0, 0)),
            pl.BlockSpec((1, D), lambda i: (0, 0)),
            pl.BlockSpec((1, D), lambda i: (0, 0)),
            pl.BlockSpec((4 * D, D), lambda i: (0, 0)),
            pl.BlockSpec((1, 4 * D), lambda i: (0, 0)),
            pl.BlockSpec((D, 4 * D), lambda i: (0, 0)),
            pl.BlockSpec((1, D), lambda i: (0, 0)),
        ],
        out_specs=pl.BlockSpec((TT, D), lambda i: (i, 0)),
        out_shape=jax.ShapeDtypeStruct((T, D), jnp.float32),
    )(xf, t, moe, Ws1.astype(jnp.bfloat16), bs1.reshape(1, 2 * D),
      Ws2.astype(jnp.bfloat16), bs2.reshape(1, D), ln2_g.reshape(1, D),
      ln2_b.reshape(1, D), Wf1.astype(jnp.bfloat16), bf1.reshape(1, 4 * D),
      Wf2.astype(jnp.bfloat16), bf2.reshape(1, D))

    return out.reshape(B, S, Dx)


# speedup vs baseline: 1.2702x; 1.2702x over previous
import functools

import jax
import jax.numpy as jnp
from jax import lax
from jax.experimental import pallas as pl
from jax.experimental.pallas import tpu as pltpu
from jax.experimental.pallas import tpu_sc as plsc

D = 768
E = 8
TEMP = 0.7
T = 2048
K = 2
NP = T * K
TT = 256
NTILES = 24
P = NTILES * TT
NW = 32
CH = NP // NW

_NC = 2
_NS = 16


def _silu(v):
    return v * jax.nn.sigmoid(v)


def _layernorm(v, g, b):
    m = jnp.mean(v, axis=-1, keepdims=True)
    var = jnp.var(v, axis=-1, keepdims=True)
    return (v - m) * lax.rsqrt(var + 1e-5) * g + b


def _ln_router_kernel(x_ref, g_ref, b_ref, wr_ref, cb_ref,
                      t_ref, idx_ref, w_ref, start_ref, te_ref):
    t = _layernorm(x_ref[...], g_ref[...], b_ref[...])
    t_ref[...] = t
    z = lax.dot_general(t, wr_ref[...], (((1,), (1,)), ((), ())),
                        preferred_element_type=jnp.float32)
    logits = lax.dot_general(z, cb_ref[...], (((1,), (1,)), ((), ())),
                             preferred_element_type=jnp.float32) / TEMP
    lm = jnp.max(logits, axis=-1, keepdims=True)
    pexp = jnp.exp(logits - lm)
    gates = pexp / jnp.sum(pexp, axis=-1, keepdims=True)
    iota_e = lax.broadcasted_iota(jnp.int32, (T, E), 1)
    m1 = jnp.max(gates, axis=-1, keepdims=True)
    i1 = jnp.min(jnp.where(gates == m1, iota_e, E), axis=-1, keepdims=True)
    masked = jnp.where(iota_e == i1, jnp.float32(-jnp.inf), gates)
    m2 = jnp.max(masked, axis=-1, keepdims=True)
    i2 = jnp.min(jnp.where(masked == m2, iota_e, E), axis=-1, keepdims=True)
    s = m1 + m2
    idx_ref[...] = jnp.concatenate([i1, i2], axis=1)
    w_ref[...] = jnp.concatenate([m1 / s, m2 / s], axis=1)
    oneh = ((iota_e == i1).astype(jnp.float32)
            + (iota_e == i2).astype(jnp.float32))
    row_w = lax.broadcasted_iota(jnp.int32, (NW, T), 0)
    col_w = lax.broadcasted_iota(jnp.int32, (NW, T), 1) // (T // NW)
    memb = (row_w == col_w).astype(jnp.float32)
    ch = lax.dot_general(memb, oneh, (((1,), (0,)), ((), ())),
                         preferred_element_type=jnp.float32)
    ls = (lax.broadcasted_iota(jnp.int32, (NW, NW), 0)
          > lax.broadcasted_iota(jnp.int32, (NW, NW), 1)).astype(jnp.float32)
    excl = lax.dot_general(ls, ch, (((1,), (0,)), ((), ())),
                           preferred_element_type=jnp.float32)
    gc = jnp.sum(ch, axis=0, keepdims=True)
    ptile = jnp.floor((gc + (TT - 1.0)) / TT)
    lse = (lax.broadcasted_iota(jnp.int32, (E, E), 0)
           < lax.broadcasted_iota(jnp.int32, (E, E), 1)).astype(jnp.float32)
    off_tile = lax.dot_general(ptile, lse, (((1,), (0,)), ((), ())),
                               preferred_element_type=jnp.float32)
    start = off_tile * TT + excl
    start_ref[...] = jnp.concatenate(
        [start, jnp.zeros((NW, 8), jnp.float32)], axis=1).astype(jnp.int32)
    off_col = jnp.reshape(off_tile, (E, 1))
    pt_col = jnp.reshape(ptile, (E, 1))
    iota_j = lax.broadcasted_iota(jnp.int32, (E, NW), 1).astype(jnp.float32)
    cover = jnp.where((iota_j >= off_col) & (iota_j < off_col + pt_col),
                      1.0, 0.0)
    iota_e2 = lax.broadcasted_iota(jnp.int32, (E, NW), 0).astype(jnp.float32)
    te_ref[...] = jnp.sum(cover * iota_e2, axis=0,
                          keepdims=True).astype(jnp.int32)


def _sc_dispatch_body(idx_hbm, start_hbm, t_hbm, slots_hbm, tsort_hbm,
                      ev_ref, rb_ref, sl_ref, rows_ref, sem):
    wid = lax.axis_index("s") * _NC + lax.axis_index("c")
    p0 = pl.multiple_of(wid * CH, CH)
    pltpu.sync_copy(idx_hbm.at[pl.ds(p0, CH)], ev_ref)
    pltpu.sync_copy(start_hbm.at[wid], rb_ref)
    iota16 = lax.iota(jnp.int32, 16)
    rb = rb_ref[...]
    for g in range(CH // 16):
        ev = ev_ref[pl.ds(g * 16, 16)]
        sl = jnp.zeros((16,), jnp.int32)
        for e in range(E):
            m = ev == e
            mi = m.astype(jnp.int32)
            pc = plsc.cumsum(mi)
            rb_e = jnp.sum(jnp.where(iota16 == e, rb, 0))
            cnt = jnp.sum(mi)
            sl = sl + jnp.where(m, rb_e + pc - 1, 0)
            rb = rb + jnp.where(iota16 == e, cnt, 0)
        sl_ref[pl.ds(g * 16, 16)] = sl
    pltpu.sync_copy(sl_ref, slots_hbm.at[pl.ds(p0, CH)])
    for g in range(CH // 16):
        tok = jnp.right_shift(p0 + g * 16 + iota16, 1)
        pltpu.async_copy(t_hbm.at[tok], rows_ref, sem).wait()
        slv = sl_ref[pl.ds(g * 16, 16)]
        pltpu.async_copy(rows_ref, tsort_hbm.at[slv], sem).wait()


def _sc_combine_body(slots_hbm, w_hbm, eo_hbm, moe_hbm,
                     sl_ref, w_ref, rows_ref, ob_ref, sem):
    wid = lax.axis_index("s") * _NC + lax.axis_index("c")
    p0 = pl.multiple_of(wid * CH, CH)
    t0 = pl.multiple_of(wid * (CH // 2), CH // 2)
    pltpu.sync_copy(slots_hbm.at[pl.ds(p0, CH)], sl_ref)
    pltpu.sync_copy(w_hbm.at[pl.ds(p0, CH)], w_ref)
    iota16 = lax.iota(jnp.int32, 16)
    for g in range(CH // 16):
        slv = sl_ref[pl.ds(g * 16, 16)]
        pltpu.async_copy(eo_hbm.at[slv], rows_ref, sem).wait()
        wv = w_ref[pl.ds(g * 16, 16)]
        for tk in range(8):
            w0 = jnp.sum(jnp.where(iota16 == 2 * tk, wv, 0.0))
            w1 = jnp.sum(jnp.where(iota16 == 2 * tk + 1, wv, 0.0))

            def body(c, carry, tk=tk, w0=w0, w1=w1):
                seg = pl.ds(c * 16, 16)
                ob_ref[tk, seg] = (rows_ref[2 * tk, seg] * w0
                                   + rows_ref[2 * tk + 1, seg] * w1)
                return carry

            lax.fori_loop(0, D // 16, body, 0)
        pltpu.sync_copy(ob_ref, moe_hbm.at[pl.ds(t0 + g * 8, 8)])


@functools.cache
def _sc_kernels():
    mesh = plsc.VectorSubcoreMesh(core_axis_name="c", subcore_axis_name="s",
                                  num_cores=_NC, num_subcores=_NS)
    params = pltpu.CompilerParams(needs_layout_passes=False)
    dispatch = pl.kernel(
        _sc_dispatch_body,
        out_type=(jax.ShapeDtypeStruct((NP,), jnp.int32),
                  jax.ShapeDtypeStruct((P, D), jnp.float32)),
        mesh=mesh,
        compiler_params=params,
        scratch_types=[pltpu.VMEM((CH,), jnp.int32),
                       pltpu.VMEM((16,), jnp.int32),
                       pltpu.VMEM((CH,), jnp.int32),
                       pltpu.VMEM((16, D), jnp.float32),
                       pltpu.SemaphoreType.DMA],
    )
    combine = pl.kernel(
        _sc_combine_body,
        out_type=jax.ShapeDtypeStruct((T, D), jnp.float32),
        mesh=mesh,
        compiler_params=params,
        scratch_types=[pltpu.VMEM((CH,), jnp.int32),
                       pltpu.VMEM((CH,), jnp.float32),
                       pltpu.VMEM((16, D), jnp.float32),
                       pltpu.VMEM((8, D), jnp.float32),
                       pltpu.SemaphoreType.DMA],
    )
    return dispatch, combine


def _grouped_kernel(te_ref, ts_ref, w1_ref, b1_ref, w2_ref, b2_ref, eo_ref):
    tb = ts_ref[...].astype(jnp.bfloat16)
    a = lax.dot_general(tb, w1_ref[0], (((1,), (1,)), ((), ())),
                        preferred_element_type=jnp.float32)
    a = _silu(a + b1_ref[0]).astype(jnp.bfloat16)
    eo = lax.dot_general(a, w2_ref[0], (((1,), (1,)), ((), ())),
                         preferred_element_type=jnp.float32)
    eo_ref[...] = eo + b2_ref[0]


def _tail_kernel(x_ref, t_ref, moe_ref, ws1_ref, bs1_ref, ws2_ref, bs2_ref,
                 g2_ref, b2_ref, wf1_ref, bf1_ref, wf2_ref, bf2_ref, o_ref):
    tb = t_ref[...].astype(jnp.bfloat16)
    s1 = lax.dot_general(tb, ws1_ref[...], (((1,), (1,)), ((), ())),
                         preferred_element_type=jnp.float32)
    s1 = _silu(s1 + bs1_ref[...]).astype(jnp.bfloat16)
    shared = lax.dot_general(s1, ws2_ref[...], (((1,), (1,)), ((), ())),
                             preferred_element_type=jnp.float32)
    shared = (shared + bs2_ref[...]) * 0.25
    hm = moe_ref[...] + shared
    f = _layernorm(hm, g2_ref[...], b2_ref[...]).astype(jnp.bfloat16)
    f1 = lax.dot_general(f, wf1_ref[...], (((1,), (1,)), ((), ())),
                         preferred_element_type=jnp.float32)
    f1 = _silu(f1 + bf1_ref[...]).astype(jnp.bfloat16)
    f2 = lax.dot_general(f1, wf2_ref[...], (((1,), (1,)), ((), ())),
                         preferred_element_type=jnp.float32)
    f2 = f2 + bf2_ref[...]
    o_ref[...] = x_ref[...] + hm + f2


def kernel(x, ln1_g, ln1_b, Wr, codebook, We1, be1, We2, be2,
           Ws1, bs1, Ws2, bs2, ln2_g, ln2_b, Wf1, bf1, Wf2, bf2):
    B, S, Dx = x.shape
    xf = x.reshape(T, Dx)
    nt = T // TT

    t, idx, wv, start, te = pl.pallas_call(
        _ln_router_kernel,
        out_shape=(jax.ShapeDtypeStruct((T, D), jnp.float32),
                   jax.ShapeDtypeStruct((T, K), jnp.int32),
                   jax.ShapeDtypeStruct((T, K), jnp.float32),
                   jax.ShapeDtypeStruct((NW, 16), jnp.int32),
                   jax.ShapeDtypeStruct((1, NW), jnp.int32)),
    )(xf, ln1_g.reshape(1, D), ln1_b.reshape(1, D), Wr, codebook)

    _sc_dispatch, _sc_combine = _sc_kernels()
    slots, tsort = _sc_dispatch(idx.reshape(NP), start, t)

    We1b = We1.astype(jnp.bfloat16)
    We2b = We2.astype(jnp.bfloat16)
    eo = pl.pallas_call(
        _grouped_kernel,
        grid_spec=pltpu.PrefetchScalarGridSpec(
            num_scalar_prefetch=1,
            grid=(NTILES,),
            in_specs=[
                pl.BlockSpec((TT, D), lambda j, te: (j, 0)),
                pl.BlockSpec((1, 4 * D, D), lambda j, te: (te[j], 0, 0)),
                pl.BlockSpec((1, 1, 4 * D), lambda j, te: (te[j], 0, 0)),
                pl.BlockSpec((1, D, 4 * D), lambda j, te: (te[j], 0, 0)),
                pl.BlockSpec((1, 1, D), lambda j, te: (te[j], 0, 0)),
            ],
            out_specs=pl.BlockSpec((TT, D), lambda j, te: (j, 0)),
        ),
        out_shape=jax.ShapeDtypeStruct((P, D), jnp.float32),
    )(te.reshape(NW), tsort, We1b, be1.reshape(E, 1, 4 * D), We2b,
      be2.reshape(E, 1, D))

    moe = _sc_combine(slots, wv.reshape(NP), eo)

    out = pl.pallas_call(
        _tail_kernel,
        grid=(nt,),
        in_specs=[
            pl.BlockSpec((TT, D), lambda i: (i, 0)),
            pl.BlockSpec((TT, D), lambda i: (i, 0)),
            pl.BlockSpec((TT, D), lambda i: (i, 0)),
            pl.BlockSpec((2 * D, D), lambda i: (0, 0)),
            pl.BlockSpec((1, 2 * D), lambda i: (0, 0)),
            pl.BlockSpec((D, 2 * D), lambda i: (0, 0)),
            pl.BlockSpec((1, D), lambda i: (0, 0)),
            pl.BlockSpec((1, D), lambda i: (0, 0)),
            pl.BlockSpec((1, D), lambda i: (0, 0)),
            pl.BlockSpec((4 * D, D), lambda i: (0, 0)),
            pl.BlockSpec((1, 4 * D), lambda i: (0, 0)),
            pl.BlockSpec((D, 4 * D), lambda i: (0, 0)),
            pl.BlockSpec((1, D), lambda i: (0, 0)),
        ],
        out_specs=pl.BlockSpec((TT, D), lambda i: (i, 0)),
        out_shape=jax.ShapeDtypeStruct((T, D), jnp.float32),
    )(xf, t, moe, Ws1.astype(jnp.bfloat16), bs1.reshape(1, 2 * D),
      Ws2.astype(jnp.bfloat16), bs2.reshape(1, D), ln2_g.reshape(1, D),
      ln2_b.reshape(1, D), Wf1.astype(jnp.bfloat16), bf1.reshape(1, 4 * D),
      Wf2.astype(jnp.bfloat16), bf2.reshape(1, D))

    return out.reshape(B, S, Dx)


# speedup vs baseline: 1.3132x; 1.0338x over previous
import functools

import jax
import jax.numpy as jnp
from jax import lax
from jax.experimental import pallas as pl
from jax.experimental.pallas import tpu as pltpu
from jax.experimental.pallas import tpu_sc as plsc

D = 768
E = 8
TEMP = 0.7
T = 2048
K = 2
NP = T * K
TT = 256
NTILES = 24
P = NTILES * TT
NW = 32
CH = NP // NW

_NC = 2
_NS = 16


def _silu(v):
    return v * jax.nn.sigmoid(v)


def _layernorm(v, g, b):
    m = jnp.mean(v, axis=-1, keepdims=True)
    var = jnp.var(v, axis=-1, keepdims=True)
    return (v - m) * lax.rsqrt(var + 1e-5) * g + b


def _ln_router_kernel(x_ref, g_ref, b_ref, wr_ref, cb_ref,
                      t_ref, idx_ref, w_ref, start_ref, te_ref):
    t = _layernorm(x_ref[...], g_ref[...], b_ref[...])
    t_ref[...] = t
    z = lax.dot_general(t, wr_ref[...], (((1,), (1,)), ((), ())),
                        preferred_element_type=jnp.float32)
    logits = lax.dot_general(z, cb_ref[...], (((1,), (1,)), ((), ())),
                             preferred_element_type=jnp.float32) / TEMP
    lm = jnp.max(logits, axis=-1, keepdims=True)
    pexp = jnp.exp(logits - lm)
    gates = pexp / jnp.sum(pexp, axis=-1, keepdims=True)
    iota_e = lax.broadcasted_iota(jnp.int32, (T, E), 1)
    m1 = jnp.max(gates, axis=-1, keepdims=True)
    i1 = jnp.min(jnp.where(gates == m1, iota_e, E), axis=-1, keepdims=True)
    masked = jnp.where(iota_e == i1, jnp.float32(-jnp.inf), gates)
    m2 = jnp.max(masked, axis=-1, keepdims=True)
    i2 = jnp.min(jnp.where(masked == m2, iota_e, E), axis=-1, keepdims=True)
    s = m1 + m2
    idx_ref[...] = jnp.concatenate([i1, i2], axis=1)
    w_ref[...] = jnp.concatenate([m1 / s, m2 / s], axis=1)
    oneh = ((iota_e == i1).astype(jnp.float32)
            + (iota_e == i2).astype(jnp.float32))
    row_w = lax.broadcasted_iota(jnp.int32, (NW, T), 0)
    col_w = lax.broadcasted_iota(jnp.int32, (NW, T), 1) // (T // NW)
    memb = (row_w == col_w).astype(jnp.float32)
    ch = lax.dot_general(memb, oneh, (((1,), (0,)), ((), ())),
                         preferred_element_type=jnp.float32)
    ls = (lax.broadcasted_iota(jnp.int32, (NW, NW), 0)
          > lax.broadcasted_iota(jnp.int32, (NW, NW), 1)).astype(jnp.float32)
    excl = lax.dot_general(ls, ch, (((1,), (0,)), ((), ())),
                           preferred_element_type=jnp.float32)
    gc = jnp.sum(ch, axis=0, keepdims=True)
    ptile = jnp.floor((gc + (TT - 1.0)) / TT)
    lse = (lax.broadcasted_iota(jnp.int32, (E, E), 0)
           < lax.broadcasted_iota(jnp.int32, (E, E), 1)).astype(jnp.float32)
    off_tile = lax.dot_general(ptile, lse, (((1,), (0,)), ((), ())),
                               preferred_element_type=jnp.float32)
    start = off_tile * TT + excl
    start_ref[...] = jnp.concatenate(
        [start, jnp.zeros((NW, 8), jnp.float32)], axis=1).astype(jnp.int32)
    off_col = jnp.reshape(off_tile, (E, 1))
    pt_col = jnp.reshape(ptile, (E, 1))
    iota_j = lax.broadcasted_iota(jnp.int32, (E, NW), 1).astype(jnp.float32)
    cover = jnp.where((iota_j >= off_col) & (iota_j < off_col + pt_col),
                      1.0, 0.0)
    iota_e2 = lax.broadcasted_iota(jnp.int32, (E, NW), 0).astype(jnp.float32)
    te_ref[...] = jnp.sum(cover * iota_e2, axis=0,
                          keepdims=True).astype(jnp.int32)


def _sc_dispatch_body(idx_hbm, start_hbm, t_hbm, slots_hbm, tsort_hbm,
                      ev_ref, rb_ref, sl_ref, rows0_ref, rows1_ref,
                      gsem0, gsem1, ssem0, ssem1):
    wid = lax.axis_index("s") * _NC + lax.axis_index("c")
    p0 = pl.multiple_of(wid * CH, CH)
    pltpu.sync_copy(idx_hbm.at[pl.ds(p0, CH)], ev_ref)
    pltpu.sync_copy(start_hbm.at[wid], rb_ref)
    iota16 = lax.iota(jnp.int32, 16)
    rb = rb_ref[...]
    for g in range(CH // 16):
        ev = ev_ref[pl.ds(g * 16, 16)]
        sl = jnp.zeros((16,), jnp.int32)
        for e in range(E):
            m = ev == e
            mi = m.astype(jnp.int32)
            pc = plsc.cumsum(mi)
            rb_e = jnp.sum(jnp.where(iota16 == e, rb, 0))
            cnt = jnp.sum(mi)
            sl = sl + jnp.where(m, rb_e + pc - 1, 0)
            rb = rb + jnp.where(iota16 == e, cnt, 0)
        sl_ref[pl.ds(g * 16, 16)] = sl
    pltpu.sync_copy(sl_ref, slots_hbm.at[pl.ds(p0, CH)])
    rows = (rows0_ref, rows1_ref)
    gsem = (gsem0, gsem1)
    ssem = (ssem0, ssem1)
    ng = CH // 16
    gh = [None, None]
    sh = [None, None]

    def tokv(g):
        return jnp.right_shift(p0 + g * 16 + iota16, 1)

    gh[0] = pltpu.async_copy(t_hbm.at[tokv(0)], rows[0], gsem[0])
    for g in range(ng):
        b = g % 2
        nb = (g + 1) % 2
        if g + 1 < ng:
            if sh[nb] is not None:
                sh[nb].wait()
            gh[nb] = pltpu.async_copy(t_hbm.at[tokv(g + 1)], rows[nb],
                                      gsem[nb])
        gh[b].wait()
        slv = sl_ref[pl.ds(g * 16, 16)]
        sh[b] = pltpu.async_copy(rows[b], tsort_hbm.at[slv], ssem[b])
    sh[0].wait()
    sh[1].wait()


def _sc_combine_body(slots_hbm, w_hbm, eo_hbm, moe_hbm,
                     sl_ref, w_ref, rows0_ref, rows1_ref, ob0_ref, ob1_ref,
                     gsem0, gsem1, wsem0, wsem1):
    wid = lax.axis_index("s") * _NC + lax.axis_index("c")
    p0 = pl.multiple_of(wid * CH, CH)
    t0 = pl.multiple_of(wid * (CH // 2), CH // 2)
    pltpu.sync_copy(slots_hbm.at[pl.ds(p0, CH)], sl_ref)
    pltpu.sync_copy(w_hbm.at[pl.ds(p0, CH)], w_ref)
    iota16 = lax.iota(jnp.int32, 16)
    rows = (rows0_ref, rows1_ref)
    obs = (ob0_ref, ob1_ref)
    gsem = (gsem0, gsem1)
    wsem = (wsem0, wsem1)
    ng = CH // 16
    gh = [None, None]
    wh = [None, None]
    gh[0] = pltpu.async_copy(eo_hbm.at[sl_ref[pl.ds(0, 16)]], rows[0],
                             gsem[0])
    for g in range(ng):
        b = g % 2
        nb = (g + 1) % 2
        if g + 1 < ng:
            gh[nb] = pltpu.async_copy(
                eo_hbm.at[sl_ref[pl.ds((g + 1) * 16, 16)]], rows[nb],
                gsem[nb])
        gh[b].wait()
        if wh[b] is not None:
            wh[b].wait()
        wv = w_ref[pl.ds(g * 16, 16)]
        for tk in range(8):
            w0 = jnp.sum(jnp.where(iota16 == 2 * tk, wv, 0.0))
            w1 = jnp.sum(jnp.where(iota16 == 2 * tk + 1, wv, 0.0))

            def body(c, carry, b=b, tk=tk, w0=w0, w1=w1):
                seg = pl.ds(c * 16, 16)
                obs[b][tk, seg] = (rows[b][2 * tk, seg] * w0
                                   + rows[b][2 * tk + 1, seg] * w1)
                return carry

            lax.fori_loop(0, D // 16, body, 0)
        wh[b] = pltpu.async_copy(obs[b], moe_hbm.at[pl.ds(t0 + g * 8, 8)],
                                 wsem[b])
    wh[0].wait()
    wh[1].wait()


@functools.cache
def _sc_kernels():
    mesh = plsc.VectorSubcoreMesh(core_axis_name="c", subcore_axis_name="s",
                                  num_cores=_NC, num_subcores=_NS)
    params = pltpu.CompilerParams(needs_layout_passes=False)
    dispatch = pl.kernel(
        _sc_dispatch_body,
        out_type=(jax.ShapeDtypeStruct((NP,), jnp.int32),
                  jax.ShapeDtypeStruct((P, D), jnp.float32)),
        mesh=mesh,
        compiler_params=params,
        scratch_types=[pltpu.VMEM((CH,), jnp.int32),
                       pltpu.VMEM((16,), jnp.int32),
                       pltpu.VMEM((CH,), jnp.int32),
                       pltpu.VMEM((16, D), jnp.float32),
                       pltpu.VMEM((16, D), jnp.float32),
                       pltpu.SemaphoreType.DMA,
                       pltpu.SemaphoreType.DMA,
                       pltpu.SemaphoreType.DMA,
                       pltpu.SemaphoreType.DMA],
    )
    combine = pl.kernel(
        _sc_combine_body,
        out_type=jax.ShapeDtypeStruct((T, D), jnp.float32),
        mesh=mesh,
        compiler_params=params,
        scratch_types=[pltpu.VMEM((CH,), jnp.int32),
                       pltpu.VMEM((CH,), jnp.float32),
                       pltpu.VMEM((16, D), jnp.float32),
                       pltpu.VMEM((16, D), jnp.float32),
                       pltpu.VMEM((8, D), jnp.float32),
                       pltpu.VMEM((8, D), jnp.float32),
                       pltpu.SemaphoreType.DMA,
                       pltpu.SemaphoreType.DMA,
                       pltpu.SemaphoreType.DMA,
                       pltpu.SemaphoreType.DMA],
    )
    return dispatch, combine


def _grouped_kernel(te_ref, ts_ref, w1_ref, b1_ref, w2_ref, b2_ref, eo_ref):
    tb = ts_ref[...].astype(jnp.bfloat16)
    a = lax.dot_general(tb, w1_ref[0], (((1,), (1,)), ((), ())),
                        preferred_element_type=jnp.float32)
    a = _silu(a + b1_ref[0]).astype(jnp.bfloat16)
    eo = lax.dot_general(a, w2_ref[0], (((1,), (1,)), ((), ())),
                         preferred_element_type=jnp.float32)
    eo_ref[...] = eo + b2_ref[0]


def _shared_kernel(t_ref, ws1_ref, bs1_ref, ws2_ref, bs2_ref, sh_ref):
    tb = t_ref[...].astype(jnp.bfloat16)
    s1 = lax.dot_general(tb, ws1_ref[...], (((1,), (1,)), ((), ())),
                         preferred_element_type=jnp.float32)
    s1 = _silu(s1 + bs1_ref[...]).astype(jnp.bfloat16)
    shared = lax.dot_general(s1, ws2_ref[...], (((1,), (1,)), ((), ())),
                             preferred_element_type=jnp.float32)
    sh_ref[...] = (shared + bs2_ref[...]) * 0.25


def _tail_kernel(x_ref, sh_ref, moe_ref,
                 g2_ref, b2_ref, wf1_ref, bf1_ref, wf2_ref, bf2_ref, o_ref):
    hm = moe_ref[...] + sh_ref[...]
    f = _layernorm(hm, g2_ref[...], b2_ref[...]).astype(jnp.bfloat16)
    f1 = lax.dot_general(f, wf1_ref[...], (((1,), (1,)), ((), ())),
                         preferred_element_type=jnp.float32)
    f1 = _silu(f1 + bf1_ref[...]).astype(jnp.bfloat16)
    f2 = lax.dot_general(f1, wf2_ref[...], (((1,), (1,)), ((), ())),
                         preferred_element_type=jnp.float32)
    f2 = f2 + bf2_ref[...]
    o_ref[...] = x_ref[...] + hm + f2


def kernel(x, ln1_g, ln1_b, Wr, codebook, We1, be1, We2, be2,
           Ws1, bs1, Ws2, bs2, ln2_g, ln2_b, Wf1, bf1, Wf2, bf2):
    B, S, Dx = x.shape
    xf = x.reshape(T, Dx)
    nt = T // TT

    t, idx, wv, start, te = pl.pallas_call(
        _ln_router_kernel,
        out_shape=(jax.ShapeDtypeStruct((T, D), jnp.float32),
                   jax.ShapeDtypeStruct((T, K), jnp.int32),
                   jax.ShapeDtypeStruct((T, K), jnp.float32),
                   jax.ShapeDtypeStruct((NW, 16), jnp.int32),
                   jax.ShapeDtypeStruct((1, NW), jnp.int32)),
    )(xf, ln1_g.reshape(1, D), ln1_b.reshape(1, D), Wr, codebook)

    _sc_dispatch, _sc_combine = _sc_kernels()
    slots, tsort = _sc_dispatch(idx.reshape(NP), start, t)

    We1b = We1.astype(jnp.bfloat16)
    We2b = We2.astype(jnp.bfloat16)
    eo = pl.pallas_call(
        _grouped_kernel,
        grid_spec=pltpu.PrefetchScalarGridSpec(
            num_scalar_prefetch=1,
            grid=(NTILES,),
            in_specs=[
                pl.BlockSpec((TT, D), lambda j, te: (j, 0)),
                pl.BlockSpec((1, 4 * D, D), lambda j, te: (te[j], 0, 0)),
                pl.BlockSpec((1, 1, 4 * D), lambda j, te: (te[j], 0, 0)),
                pl.BlockSpec((1, D, 4 * D), lambda j, te: (te[j], 0, 0)),
                pl.BlockSpec((1, 1, D), lambda j, te: (te[j], 0, 0)),
            ],
            out_specs=pl.BlockSpec((TT, D), lambda j, te: (j, 0)),
        ),
        out_shape=jax.ShapeDtypeStruct((P, D), jnp.float32),
    )(te.reshape(NW), tsort, We1b, be1.reshape(E, 1, 4 * D), We2b,
      be2.reshape(E, 1, D))

    sh = pl.pallas_call(
        _shared_kernel,
        grid=(nt,),
        in_specs=[
            pl.BlockSpec((TT, D), lambda i: (i, 0)),
            pl.BlockSpec((2 * D, D), lambda i: (0, 0)),
            pl.BlockSpec((1, 2 * D), lambda i: (0, 0)),
            pl.BlockSpec((D, 2 * D), lambda i: (0, 0)),
            pl.BlockSpec((1, D), lambda i: (0, 0)),
        ],
        out_specs=pl.BlockSpec((TT, D), lambda i: (i, 0)),
        out_shape=jax.ShapeDtypeStruct((T, D), jnp.float32),
    )(t, Ws1.astype(jnp.bfloat16), bs1.reshape(1, 2 * D),
      Ws2.astype(jnp.bfloat16), bs2.reshape(1, D))

    moe = _sc_combine(slots, wv.reshape(NP), eo)

    out = pl.pallas_call(
        _tail_kernel,
        grid=(nt,),
        in_specs=[
            pl.BlockSpec((TT, D), lambda i: (i, 0)),
            pl.BlockSpec((TT, D), lambda i: (i, 0)),
            pl.BlockSpec((TT, D), lambda i: (i, 0)),
            pl.BlockSpec((1, D), lambda i: (0, 0)),
            pl.BlockSpec((1, D), lambda i: (0, 0)),
            pl.BlockSpec((4 * D, D), lambda i: (0, 0)),
            pl.BlockSpec((1, 4 * D), lambda i: (0, 0)),
            pl.BlockSpec((D, 4 * D), lambda i: (0, 0)),
            pl.BlockSpec((1, D), lambda i: (0, 0)),
        ],
        out_specs=pl.BlockSpec((TT, D), lambda i: (i, 0)),
        out_shape=jax.ShapeDtypeStruct((T, D), jnp.float32),
    )(xf, sh, moe, ln2_g.reshape(1, D),
      ln2_b.reshape(1, D), Wf1.astype(jnp.bfloat16), bf1.reshape(1, 4 * D),
      Wf2.astype(jnp.bfloat16), bf2.reshape(1, D))

    return out.reshape(B, S, Dx)


# speedup vs baseline: 1.5787x; 1.2022x over previous
import functools

import jax
import jax.numpy as jnp
from jax import lax
from jax.experimental import pallas as pl
from jax.experimental.pallas import tpu as pltpu
from jax.experimental.pallas import tpu_sc as plsc

D = 768
E = 8
TEMP = 0.7
T = 2048
K = 2
NP = T * K
TT = 256
NTILES = 24
P = NTILES * TT
NW = 32
CH = NP // NW

_NC = 2
_NS = 16


def _silu(v):
    return v * jax.nn.sigmoid(v)


def _layernorm(v, g, b):
    m = jnp.mean(v, axis=-1, keepdims=True)
    var = jnp.var(v, axis=-1, keepdims=True)
    return (v - m) * lax.rsqrt(var + 1e-5) * g + b


def _ln_router_kernel(x_ref, g_ref, b_ref, wr_ref, cb_ref,
                      t_ref, idx_ref, w_ref, start_ref, te_ref):
    t = _layernorm(x_ref[...], g_ref[...], b_ref[...])
    t_ref[...] = t
    z = lax.dot_general(t, wr_ref[...], (((1,), (1,)), ((), ())),
                        preferred_element_type=jnp.float32)
    logits = lax.dot_general(z, cb_ref[...], (((1,), (1,)), ((), ())),
                             preferred_element_type=jnp.float32) / TEMP
    lm = jnp.max(logits, axis=-1, keepdims=True)
    pexp = jnp.exp(logits - lm)
    gates = pexp / jnp.sum(pexp, axis=-1, keepdims=True)
    iota_e = lax.broadcasted_iota(jnp.int32, (T, E), 1)
    m1 = jnp.max(gates, axis=-1, keepdims=True)
    i1 = jnp.min(jnp.where(gates == m1, iota_e, E), axis=-1, keepdims=True)
    masked = jnp.where(iota_e == i1, jnp.float32(-jnp.inf), gates)
    m2 = jnp.max(masked, axis=-1, keepdims=True)
    i2 = jnp.min(jnp.where(masked == m2, iota_e, E), axis=-1, keepdims=True)
    s = m1 + m2
    idx_ref[...] = jnp.concatenate([i1, i2], axis=1)
    w_ref[...] = jnp.concatenate([m1 / s, m2 / s], axis=1)
    oneh = ((iota_e == i1).astype(jnp.float32)
            + (iota_e == i2).astype(jnp.float32))
    row_w = lax.broadcasted_iota(jnp.int32, (NW, T), 0)
    col_w = lax.broadcasted_iota(jnp.int32, (NW, T), 1) // (T // NW)
    memb = (row_w == col_w).astype(jnp.float32)
    ch = lax.dot_general(memb, oneh, (((1,), (0,)), ((), ())),
                         preferred_element_type=jnp.float32)
    ls = (lax.broadcasted_iota(jnp.int32, (NW, NW), 0)
          > lax.broadcasted_iota(jnp.int32, (NW, NW), 1)).astype(jnp.float32)
    excl = lax.dot_general(ls, ch, (((1,), (0,)), ((), ())),
                           preferred_element_type=jnp.float32)
    gc = jnp.sum(ch, axis=0, keepdims=True)
    ptile = jnp.floor((gc + (TT - 1.0)) / TT)
    lse = (lax.broadcasted_iota(jnp.int32, (E, E), 0)
           < lax.broadcasted_iota(jnp.int32, (E, E), 1)).astype(jnp.float32)
    off_tile = lax.dot_general(ptile, lse, (((1,), (0,)), ((), ())),
                               preferred_element_type=jnp.float32)
    start = off_tile * TT + excl
    start_ref[...] = jnp.concatenate(
        [start, jnp.zeros((NW, 8), jnp.float32)], axis=1).astype(jnp.int32)
    off_col = jnp.reshape(off_tile, (E, 1))
    pt_col = jnp.reshape(ptile, (E, 1))
    iota_j = lax.broadcasted_iota(jnp.int32, (E, NW), 1).astype(jnp.float32)
    cover = jnp.where((iota_j >= off_col) & (iota_j < off_col + pt_col),
                      1.0, 0.0)
    iota_e2 = lax.broadcasted_iota(jnp.int32, (E, NW), 0).astype(jnp.float32)
    te_ref[...] = jnp.sum(cover * iota_e2, axis=0,
                          keepdims=True).astype(jnp.int32)


def _sc_dispatch_body(idx_hbm, start_hbm, t_hbm, slots_hbm, tsort_hbm,
                      ev_ref, rb_ref, sl_ref, rows0_ref, rows1_ref,
                      gsem0, gsem1, ssem0, ssem1):
    wid = lax.axis_index("s") * _NC + lax.axis_index("c")
    p0 = pl.multiple_of(wid * CH, CH)
    pltpu.sync_copy(idx_hbm.at[pl.ds(p0, CH)], ev_ref)
    pltpu.sync_copy(start_hbm.at[wid], rb_ref)
    iota16 = lax.iota(jnp.int32, 16)
    rb = rb_ref[...]
    for g in range(CH // 16):
        ev = ev_ref[pl.ds(g * 16, 16)]
        sl = jnp.zeros((16,), jnp.int32)
        for e in range(E):
            m = ev == e
            mi = m.astype(jnp.int32)
            pc = plsc.cumsum(mi)
            rb_e = jnp.sum(jnp.where(iota16 == e, rb, 0))
            cnt = jnp.sum(mi)
            sl = sl + jnp.where(m, rb_e + pc - 1, 0)
            rb = rb + jnp.where(iota16 == e, cnt, 0)
        sl_ref[pl.ds(g * 16, 16)] = sl
    pltpu.sync_copy(sl_ref, slots_hbm.at[pl.ds(p0, CH)])
    rows = (rows0_ref, rows1_ref)
    gsem = (gsem0, gsem1)
    ssem = (ssem0, ssem1)
    ng = CH // 16
    gh = [None, None]
    sh = [None, None]

    def tokv(g):
        return jnp.right_shift(p0 + g * 16 + iota16, 1)

    gh[0] = pltpu.async_copy(t_hbm.at[tokv(0)], rows[0], gsem[0])
    for g in range(ng):
        b = g % 2
        nb = (g + 1) % 2
        if g + 1 < ng:
            if sh[nb] is not None:
                sh[nb].wait()
            gh[nb] = pltpu.async_copy(t_hbm.at[tokv(g + 1)], rows[nb],
                                      gsem[nb])
        gh[b].wait()
        slv = sl_ref[pl.ds(g * 16, 16)]
        sh[b] = pltpu.async_copy(rows[b], tsort_hbm.at[slv], ssem[b])
    sh[0].wait()
    sh[1].wait()


def _sc_combine_body(slots_hbm, w_hbm, eo_hbm, moe_hbm,
                     sl_ref, w_ref, rows0_ref, rows1_ref, ob0_ref, ob1_ref,
                     gsem0, gsem1, wsem0, wsem1):
    wid = lax.axis_index("s") * _NC + lax.axis_index("c")
    p0 = pl.multiple_of(wid * CH, CH)
    t0 = pl.multiple_of(wid * (CH // 2), CH // 2)
    pltpu.sync_copy(slots_hbm.at[pl.ds(p0, CH)], sl_ref)
    pltpu.sync_copy(w_hbm.at[pl.ds(p0, CH)], w_ref)
    iota16 = lax.iota(jnp.int32, 16)
    rows = (rows0_ref, rows1_ref)
    obs = (ob0_ref, ob1_ref)
    gsem = (gsem0, gsem1)
    wsem = (wsem0, wsem1)
    ng = CH // 16
    gh = [None, None]
    wh = [None, None]
    gh[0] = pltpu.async_copy(eo_hbm.at[sl_ref[pl.ds(0, 16)]], rows[0],
                             gsem[0])
    for g in range(ng):
        b = g % 2
        nb = (g + 1) % 2
        if g + 1 < ng:
            gh[nb] = pltpu.async_copy(
                eo_hbm.at[sl_ref[pl.ds((g + 1) * 16, 16)]], rows[nb],
                gsem[nb])
        gh[b].wait()
        if wh[b] is not None:
            wh[b].wait()
        wv = w_ref[pl.ds(g * 16, 16)]
        for tk in range(8):
            w0 = jnp.sum(jnp.where(iota16 == 2 * tk, wv, 0.0))
            w1 = jnp.sum(jnp.where(iota16 == 2 * tk + 1, wv, 0.0))

            def body(c, carry, b=b, tk=tk, w0=w0, w1=w1):
                seg = pl.ds(c * 16, 16)
                obs[b][tk, seg] = (rows[b][2 * tk, seg] * w0
                                   + rows[b][2 * tk + 1, seg] * w1)
                return carry

            lax.fori_loop(0, D // 16, body, 0)
        wh[b] = pltpu.async_copy(obs[b], moe_hbm.at[pl.ds(t0 + g * 8, 8)],
                                 wsem[b])
    wh[0].wait()
    wh[1].wait()


@functools.cache
def _sc_kernels():
    mesh = plsc.VectorSubcoreMesh(core_axis_name="c", subcore_axis_name="s",
                                  num_cores=_NC, num_subcores=_NS)
    params = pltpu.CompilerParams(needs_layout_passes=False)
    dispatch = pl.kernel(
        _sc_dispatch_body,
        out_type=(jax.ShapeDtypeStruct((NP,), jnp.int32),
                  jax.ShapeDtypeStruct((P, D), jnp.float32)),
        mesh=mesh,
        compiler_params=params,
        scratch_types=[pltpu.VMEM((CH,), jnp.int32),
                       pltpu.VMEM((16,), jnp.int32),
                       pltpu.VMEM((CH,), jnp.int32),
                       pltpu.VMEM((16, D), jnp.float32),
                       pltpu.VMEM((16, D), jnp.float32),
                       pltpu.SemaphoreType.DMA,
                       pltpu.SemaphoreType.DMA,
                       pltpu.SemaphoreType.DMA,
                       pltpu.SemaphoreType.DMA],
    )
    combine = pl.kernel(
        _sc_combine_body,
        out_type=jax.ShapeDtypeStruct((T, D), jnp.float32),
        mesh=mesh,
        compiler_params=params,
        scratch_types=[pltpu.VMEM((CH,), jnp.int32),
                       pltpu.VMEM((CH,), jnp.float32),
                       pltpu.VMEM((16, D), jnp.float32),
                       pltpu.VMEM((16, D), jnp.float32),
                       pltpu.VMEM((8, D), jnp.float32),
                       pltpu.VMEM((8, D), jnp.float32),
                       pltpu.SemaphoreType.DMA,
                       pltpu.SemaphoreType.DMA,
                       pltpu.SemaphoreType.DMA,
                       pltpu.SemaphoreType.DMA],
    )
    return dispatch, combine


def _grouped_kernel(te_ref, ts_ref, w1_ref, b1_ref, w2_ref, b2_ref, eo_ref,
                    w1b_ref, w2b_ref):
    j = pl.program_id(0)
    changed = (j == 0) | (te_ref[j] != te_ref[jnp.maximum(j - 1, 0)])

    @pl.when(changed)
    def _():
        w1b_ref[...] = w1_ref[0].astype(jnp.bfloat16)
        w2b_ref[...] = w2_ref[0].astype(jnp.bfloat16)

    tb = ts_ref[...].astype(jnp.bfloat16)
    a = lax.dot_general(tb, w1b_ref[...], (((1,), (1,)), ((), ())),
                        preferred_element_type=jnp.float32)
    a = _silu(a + b1_ref[0]).astype(jnp.bfloat16)
    eo = lax.dot_general(a, w2b_ref[...], (((1,), (1,)), ((), ())),
                         preferred_element_type=jnp.float32)
    eo_ref[...] = eo + b2_ref[0]


def _shared_kernel(t_ref, ws1_ref, bs1_ref, ws2_ref, bs2_ref, sh_ref,
                   ws1b_ref, ws2b_ref):
    @pl.when(pl.program_id(0) == 0)
    def _():
        ws1b_ref[...] = ws1_ref[...].astype(jnp.bfloat16)
        ws2b_ref[...] = ws2_ref[...].astype(jnp.bfloat16)

    tb = t_ref[...].astype(jnp.bfloat16)
    s1 = lax.dot_general(tb, ws1b_ref[...], (((1,), (1,)), ((), ())),
                         preferred_element_type=jnp.float32)
    s1 = _silu(s1 + bs1_ref[...]).astype(jnp.bfloat16)
    shared = lax.dot_general(s1, ws2b_ref[...], (((1,), (1,)), ((), ())),
                             preferred_element_type=jnp.float32)
    sh_ref[...] = (shared + bs2_ref[...]) * 0.25


def _tail_kernel(x_ref, sh_ref, moe_ref,
                 g2_ref, b2_ref, wf1_ref, bf1_ref, wf2_ref, bf2_ref, o_ref,
                 wf1b_ref, wf2b_ref):
    @pl.when(pl.program_id(0) == 0)
    def _():
        wf1b_ref[...] = wf1_ref[...].astype(jnp.bfloat16)
        wf2b_ref[...] = wf2_ref[...].astype(jnp.bfloat16)

    hm = moe_ref[...] + sh_ref[...]
    f = _layernorm(hm, g2_ref[...], b2_ref[...]).astype(jnp.bfloat16)
    f1 = lax.dot_general(f, wf1b_ref[...], (((1,), (1,)), ((), ())),
                         preferred_element_type=jnp.float32)
    f1 = _silu(f1 + bf1_ref[...]).astype(jnp.bfloat16)
    f2 = lax.dot_general(f1, wf2b_ref[...], (((1,), (1,)), ((), ())),
                         preferred_element_type=jnp.float32)
    f2 = f2 + bf2_ref[...]
    o_ref[...] = x_ref[...] + hm + f2


def kernel(x, ln1_g, ln1_b, Wr, codebook, We1, be1, We2, be2,
           Ws1, bs1, Ws2, bs2, ln2_g, ln2_b, Wf1, bf1, Wf2, bf2):
    B, S, Dx = x.shape
    xf = x.reshape(T, Dx)
    nt = T // TT

    t, idx, wv, start, te = pl.pallas_call(
        _ln_router_kernel,
        out_shape=(jax.ShapeDtypeStruct((T, D), jnp.float32),
                   jax.ShapeDtypeStruct((T, K), jnp.int32),
                   jax.ShapeDtypeStruct((T, K), jnp.float32),
                   jax.ShapeDtypeStruct((NW, 16), jnp.int32),
                   jax.ShapeDtypeStruct((1, NW), jnp.int32)),
    )(xf, ln1_g.reshape(1, D), ln1_b.reshape(1, D), Wr, codebook)

    _sc_dispatch, _sc_combine = _sc_kernels()
    slots, tsort = _sc_dispatch(idx.reshape(NP), start, t)

    eo = pl.pallas_call(
        _grouped_kernel,
        grid_spec=pltpu.PrefetchScalarGridSpec(
            num_scalar_prefetch=1,
            grid=(NTILES,),
            in_specs=[
                pl.BlockSpec((TT, D), lambda j, te: (j, 0)),
                pl.BlockSpec((1, 4 * D, D), lambda j, te: (te[j], 0, 0)),
                pl.BlockSpec((1, 1, 4 * D), lambda j, te: (te[j], 0, 0)),
                pl.BlockSpec((1, D, 4 * D), lambda j, te: (te[j], 0, 0)),
                pl.BlockSpec((1, 1, D), lambda j, te: (te[j], 0, 0)),
            ],
            out_specs=pl.BlockSpec((TT, D), lambda j, te: (j, 0)),
            scratch_shapes=[pltpu.VMEM((4 * D, D), jnp.bfloat16),
                            pltpu.VMEM((D, 4 * D), jnp.bfloat16)],
        ),
        out_shape=jax.ShapeDtypeStruct((P, D), jnp.float32),
    )(te.reshape(NW), tsort, We1, be1.reshape(E, 1, 4 * D), We2,
      be2.reshape(E, 1, D))

    sh = pl.pallas_call(
        _shared_kernel,
        grid=(nt,),
        in_specs=[
            pl.BlockSpec((TT, D), lambda i: (i, 0)),
            pl.BlockSpec((2 * D, D), lambda i: (0, 0)),
            pl.BlockSpec((1, 2 * D), lambda i: (0, 0)),
            pl.BlockSpec((D, 2 * D), lambda i: (0, 0)),
            pl.BlockSpec((1, D), lambda i: (0, 0)),
        ],
        out_specs=pl.BlockSpec((TT, D), lambda i: (i, 0)),
        out_shape=jax.ShapeDtypeStruct((T, D), jnp.float32),
        scratch_shapes=[pltpu.VMEM((2 * D, D), jnp.bfloat16),
                        pltpu.VMEM((D, 2 * D), jnp.bfloat16)],
    )(t, Ws1, bs1.reshape(1, 2 * D), Ws2, bs2.reshape(1, D))

    moe = _sc_combine(slots, wv.reshape(NP), eo)

    out = pl.pallas_call(
        _tail_kernel,
        grid=(nt,),
        in_specs=[
            pl.BlockSpec((TT, D), lambda i: (i, 0)),
            pl.BlockSpec((TT, D), lambda i: (i, 0)),
            pl.BlockSpec((TT, D), lambda i: (i, 0)),
            pl.BlockSpec((1, D), lambda i: (0, 0)),
            pl.BlockSpec((1, D), lambda i: (0, 0)),
            pl.BlockSpec((4 * D, D), lambda i: (0, 0)),
            pl.BlockSpec((1, 4 * D), lambda i: (0, 0)),
            pl.BlockSpec((D, 4 * D), lambda i: (0, 0)),
            pl.BlockSpec((1, D), lambda i: (0, 0)),
        ],
        out_specs=pl.BlockSpec((TT, D), lambda i: (i, 0)),
        out_shape=jax.ShapeDtypeStruct((T, D), jnp.float32),
        scratch_shapes=[pltpu.VMEM((4 * D, D), jnp.bfloat16),
                        pltpu.VMEM((D, 4 * D), jnp.bfloat16)],
    )(xf, sh, moe, ln2_g.reshape(1, D),
      ln2_b.reshape(1, D), Wf1, bf1.reshape(1, 4 * D),
      Wf2, bf2.reshape(1, D))

    return out.reshape(B, S, Dx)
